# baseline (speedup 1.0000x reference)
import functools

import numpy as np
import jax
import jax.numpy as jnp
from jax import lax
from jax.experimental import pallas as pl
from jax.experimental.pallas import tpu as pltpu
from jax.experimental.pallas import tpu_sc as plsc

VOCAB = 100000
EMBD = 128
MAX_LEN = 8192
BATCH = 4
SEQ = 8192
TOK = BATCH * SEQ
NW = 32
SRANGE = SEQ // NW
PER = BATCH * SRANGE
CHUNK = 128
NCHUNK = PER // CHUNK
HALVES = SRANGE // CHUNK
LANES = 16


def _position_table() -> np.ndarray:
    pos = np.arange(MAX_LEN, dtype=np.float64)[:, None]
    div = np.arange(0, EMBD, 2, dtype=np.float64)[None, :]
    m = (pos / (10000.0 ** (div / EMBD))).astype(np.float32)
    return np.concatenate([np.sin(m), np.cos(m)], axis=-1).astype(np.float32)


_POS_FLAT = _position_table().reshape(-1)

_MESH = plsc.VectorSubcoreMesh(core_axis_name="c", subcore_axis_name="s")


@functools.partial(
    pl.kernel,
    mesh=_MESH,
    out_type=[
        jax.ShapeDtypeStruct((BATCH, SEQ, EMBD), jnp.float32),
        jax.ShapeDtypeStruct((NW, BATCH * LANES), jnp.int32),
    ],
    scratch_types=[
        pltpu.VMEM((PER,), jnp.int32),
        pltpu.VMEM((SRANGE * EMBD,), jnp.float32),
        pltpu.VMEM((CHUNK, EMBD), jnp.float32),
        pltpu.VMEM((CHUNK, EMBD), jnp.float32),
        pltpu.VMEM((CHUNK, EMBD), jnp.float32),
        pltpu.VMEM((BATCH * LANES,), jnp.int32),
        pltpu.SemaphoreType.DMA,
        pltpu.SemaphoreType.DMA,
        pltpu.SemaphoreType.DMA,
        pltpu.SemaphoreType.DMA,
        pltpu.SemaphoreType.DMA,
        pltpu.SemaphoreType.DMA,
    ],
)
def _embed_sc(ids_h, tab_h, pos_h, out_h, cnt_h,
              idx_v, pbuf, gbuf0, gbuf1, gbuf2, cnt_v,
              psem, gsem0, gsem1, gsem2, osem0, osem1):
    wid = lax.axis_index("s") * 2 + lax.axis_index("c")
    sbase = wid * SRANGE

    pcp = pltpu.async_copy(pos_h.at[pl.ds(sbase * EMBD, SRANGE * EMBD)],
                           pbuf, psem)

    for b in range(BATCH):
        pltpu.sync_copy(ids_h.at[b, pl.ds(sbase, SRANGE)],
                        idx_v.at[pl.ds(b * SRANGE, SRANGE)])

    gbufs = (gbuf0, gbuf1, gbuf2)
    gsems = (gsem0, gsem1, gsem2)
    osems = (osem0, osem1)

    def issue(c):
        return pltpu.async_copy(
            tab_h.at[idx_v.at[pl.ds(c * CHUNK, CHUNK)]],
            gbufs[c % 3], gsems[c % 3])

    inflight = [None] * NCHUNK
    ostores = [None] * NCHUNK
    inflight[0] = issue(0)
    inflight[1] = issue(1)

    for b in range(BATCH):
        def count_body(t, acc):
            v = idx_v[pl.ds(b * SRANGE + t * LANES, LANES)]
            return acc + jnp.where(v == 1, 1, 0).astype(jnp.int32)

        acc = lax.fori_loop(0, SRANGE // LANES, count_body,
                            jnp.zeros((LANES,), jnp.int32))
        cnt_v[pl.ds(b * LANES, LANES)] = acc
    pltpu.sync_copy(cnt_v, cnt_h.at[wid])

    pcp.wait()
    for c in range(NCHUNK):
        if c + 2 < NCHUNK:
            if c >= 1:
                ostores[c - 1].wait()
            inflight[c + 2] = issue(c + 2)
        inflight[c].wait()
        gbuf = gbufs[c % 3]
        poff = (c % HALVES) * CHUNK

        @plsc.parallel_loop(0, CHUNK, 1, unroll=4)
        def _(r):
            pb = (poff + r) * EMBD
            for j in range(EMBD // LANES):
                vec = pbuf[pl.ds(pb + j * LANES, LANES)]
                plsc.addupdate(gbuf.at[r, pl.ds(j * LANES, LANES)], vec)
        ostores[c] = pltpu.async_copy(
            gbuf, out_h.at[c // HALVES, pl.ds(sbase + poff, CHUNK)],
            osems[c % 2])
    ostores[NCHUNK - 3].wait()
    ostores[NCHUNK - 2].wait()
    ostores[NCHUNK - 1].wait()


def kernel(ids, word_embedding):
    pos = jnp.asarray(_POS_FLAT)
    out, cnt = _embed_sc(ids, word_embedding, pos)
    padding_len = cnt.reshape(NW, BATCH, LANES).sum(axis=(0, 2))
    return (out, padding_len)


# speedup vs baseline: 1.0491x; 1.0491x over previous
import functools

import numpy as np
import jax
import jax.numpy as jnp
from jax import lax
from jax.experimental import pallas as pl
from jax.experimental.pallas import tpu as pltpu
from jax.experimental.pallas import tpu_sc as plsc

VOCAB = 100000
EMBD = 128
MAX_LEN = 8192
BATCH = 4
SEQ = 8192
TOK = BATCH * SEQ
NW = 32
SRANGE = SEQ // NW
PER = BATCH * SRANGE
CHUNK = 128
NCHUNK = PER // CHUNK
HALVES = SRANGE // CHUNK
LANES = 16


def _position_table() -> np.ndarray:
    pos = np.arange(MAX_LEN, dtype=np.float64)[:, None]
    div = np.arange(0, EMBD, 2, dtype=np.float64)[None, :]
    m = (pos / (10000.0 ** (div / EMBD))).astype(np.float32)
    return np.concatenate([np.sin(m), np.cos(m)], axis=-1).astype(np.float32)


_POS_FLAT = _position_table().reshape(-1)
_POS_DEV = None


def _pos_dev():
    global _POS_DEV
    if _POS_DEV is None:
        _POS_DEV = jax.device_put(_POS_FLAT)
    return _POS_DEV

_MESH = plsc.VectorSubcoreMesh(core_axis_name="c", subcore_axis_name="s")


@functools.partial(
    pl.kernel,
    mesh=_MESH,
    out_type=[
        jax.ShapeDtypeStruct((BATCH, SEQ, EMBD), jnp.float32),
        jax.ShapeDtypeStruct((NW, BATCH * LANES), jnp.int32),
    ],
    scratch_types=[
        pltpu.VMEM((PER,), jnp.int32),
        pltpu.VMEM((SRANGE * EMBD,), jnp.float32),
        pltpu.VMEM((CHUNK, EMBD), jnp.float32),
        pltpu.VMEM((CHUNK, EMBD), jnp.float32),
        pltpu.VMEM((CHUNK, EMBD), jnp.float32),
        pltpu.VMEM((BATCH * LANES,), jnp.int32),
        pltpu.SemaphoreType.DMA,
        pltpu.SemaphoreType.DMA,
        pltpu.SemaphoreType.DMA,
        pltpu.SemaphoreType.DMA,
        pltpu.SemaphoreType.DMA,
        pltpu.SemaphoreType.DMA,
    ],
)
def _embed_sc(ids_h, tab_h, pos_h, out_h, cnt_h,
              idx_v, pbuf, gbuf0, gbuf1, gbuf2, cnt_v,
              psem, gsem0, gsem1, gsem2, osem0, osem1):
    wid = lax.axis_index("s") * 2 + lax.axis_index("c")
    sbase = wid * SRANGE

    pcp = pltpu.async_copy(pos_h.at[pl.ds(sbase * EMBD, SRANGE * EMBD)],
                           pbuf, psem)

    for b in range(BATCH):
        pltpu.sync_copy(ids_h.at[b, pl.ds(sbase, SRANGE)],
                        idx_v.at[pl.ds(b * SRANGE, SRANGE)])

    gbufs = (gbuf0, gbuf1, gbuf2)
    gsems = (gsem0, gsem1, gsem2)
    osems = (osem0, osem1)

    def issue(c):
        return pltpu.async_copy(
            tab_h.at[idx_v.at[pl.ds(c * CHUNK, CHUNK)]],
            gbufs[c % 3], gsems[c % 3])

    inflight = [None] * NCHUNK
    ostores = [None] * NCHUNK
    inflight[0] = issue(0)
    inflight[1] = issue(1)

    for b in range(BATCH):
        def count_body(t, acc):
            v = idx_v[pl.ds(b * SRANGE + t * LANES, LANES)]
            return acc + jnp.where(v == 1, 1, 0).astype(jnp.int32)

        acc = lax.fori_loop(0, SRANGE // LANES, count_body,
                            jnp.zeros((LANES,), jnp.int32))
        cnt_v[pl.ds(b * LANES, LANES)] = acc
    pltpu.sync_copy(cnt_v, cnt_h.at[wid])

    pcp.wait()
    for c in range(NCHUNK):
        if c + 2 < NCHUNK:
            if c >= 1:
                ostores[c - 1].wait()
            inflight[c + 2] = issue(c + 2)
        inflight[c].wait()
        gbuf = gbufs[c % 3]
        poff = (c % HALVES) * CHUNK

        def add_body(r, _):
            pb = (poff + r) * EMBD
            for j in range(EMBD // LANES):
                vec = pbuf[pl.ds(pb + j * LANES, LANES)]
                plsc.addupdate(gbuf.at[r, pl.ds(j * LANES, LANES)], vec)
            return 0

        lax.fori_loop(0, CHUNK, add_body, 0)
        ostores[c] = pltpu.async_copy(
            gbuf, out_h.at[c // HALVES, pl.ds(sbase + poff, CHUNK)],
            osems[c % 2])
    ostores[NCHUNK - 3].wait()
    ostores[NCHUNK - 2].wait()
    ostores[NCHUNK - 1].wait()


def kernel(ids, word_embedding):
    pos = _pos_dev()
    out, cnt = _embed_sc(ids, word_embedding, pos)
    padding_len = cnt.reshape(NW, BATCH, LANES).sum(axis=(0, 2))
    return (out, padding_len)


# speedup vs baseline: 1.0908x; 1.0398x over previous
import functools

import numpy as np
import jax
import jax.numpy as jnp
from jax import lax
from jax.experimental import pallas as pl
from jax.experimental.pallas import tpu as pltpu
from jax.experimental.pallas import tpu_sc as plsc

VOCAB = 100000
EMBD = 128
MAX_LEN = 8192
BATCH = 4
SEQ = 8192
TOK = BATCH * SEQ
NW = 32
SRANGE = SEQ // NW
PER = BATCH * SRANGE
CHUNK = 128
NCHUNK = PER // CHUNK
HALVES = SRANGE // CHUNK
LANES = 16
NBUF = 4


def _position_table() -> np.ndarray:
    pos = np.arange(MAX_LEN, dtype=np.float64)[:, None]
    div = np.arange(0, EMBD, 2, dtype=np.float64)[None, :]
    m = (pos / (10000.0 ** (div / EMBD))).astype(np.float32)
    return np.concatenate([np.sin(m), np.cos(m)], axis=-1).astype(np.float32)


_POS_FLAT = _position_table().reshape(-1)

_MESH = plsc.VectorSubcoreMesh(core_axis_name="c", subcore_axis_name="s")


@functools.partial(
    pl.kernel,
    mesh=_MESH,
    out_type=[
        jax.ShapeDtypeStruct((BATCH, SEQ, EMBD), jnp.float32),
        jax.ShapeDtypeStruct((NW, BATCH * LANES), jnp.int32),
    ],
    scratch_types=[
        pltpu.VMEM((BATCH, SRANGE), jnp.int32),
        pltpu.VMEM((SRANGE * EMBD,), jnp.float32),
        pltpu.VMEM((NBUF, CHUNK, EMBD), jnp.float32),
        pltpu.VMEM((BATCH * LANES,), jnp.int32),
        pltpu.SemaphoreType.DMA,
        pltpu.SemaphoreType.DMA,
        pltpu.SemaphoreType.DMA,
        pltpu.SemaphoreType.DMA,
        pltpu.SemaphoreType.DMA,
        pltpu.SemaphoreType.DMA,
        pltpu.SemaphoreType.DMA,
    ],
)
def _embed_sc(ids_h, tab_h, pos_h, out_h, cnt_h,
              idx_v, pbuf, gring, cnt_v,
              psem, gsem0, gsem1, gsem2, gsem3, osem0, osem1):
    wid = lax.axis_index("s") * 2 + lax.axis_index("c")
    sbase = wid * SRANGE

    pcp = pltpu.async_copy(pos_h.at[pl.ds(sbase * EMBD, SRANGE * EMBD)],
                           pbuf, psem)

    pltpu.sync_copy(ids_h.at[:, pl.ds(sbase, SRANGE)], idx_v)

    gsems = (gsem0, gsem1, gsem2, gsem3)
    osems = (osem0, osem1)

    def issue(c):
        return pltpu.async_copy(
            tab_h.at[idx_v.at[c // HALVES,
                              pl.ds((c % HALVES) * CHUNK, CHUNK)]],
            gring.at[c % NBUF], gsems[c % NBUF])

    inflight = [None] * NCHUNK
    ostores = [None] * NCHUNK
    inflight[0] = issue(0)
    inflight[1] = issue(1)
    inflight[2] = issue(2)

    for b in range(BATCH):
        def count_body(t, acc):
            v = idx_v[b, pl.ds(t * LANES, LANES)]
            return acc + jnp.where(v == 1, 1, 0).astype(jnp.int32)

        acc = lax.fori_loop(0, SRANGE // LANES, count_body,
                            jnp.zeros((LANES,), jnp.int32))
        cnt_v[pl.ds(b * LANES, LANES)] = acc
    pltpu.sync_copy(cnt_v, cnt_h.at[wid])

    pcp.wait()
    for c in range(NCHUNK):
        if c + 3 < NCHUNK:
            if c >= 1:
                ostores[c - 1].wait()
            inflight[c + 3] = issue(c + 3)
        inflight[c].wait()
        gbuf = gring.at[c % NBUF]
        poff = (c % HALVES) * CHUNK

        def add_body(r, _):
            pb = (poff + r) * EMBD
            for j in range(EMBD // LANES):
                vec = pbuf[pl.ds(pb + j * LANES, LANES)]
                plsc.addupdate(gbuf.at[r, pl.ds(j * LANES, LANES)], vec)
            return 0

        lax.fori_loop(0, CHUNK, add_body, 0)
        ostores[c] = pltpu.async_copy(
            gbuf, out_h.at[c // HALVES, pl.ds(sbase + poff, CHUNK)],
            osems[c % 2])
    for c in range(NCHUNK - 4, NCHUNK):
        if ostores[c] is not None:
            ostores[c].wait()


def kernel(ids, word_embedding):
    pos = jnp.asarray(_POS_FLAT)
    out, cnt = _embed_sc(ids, word_embedding, pos)
    padding_len = cnt.reshape(NW, BATCH, LANES).sum(axis=(0, 2))
    return (out, padding_len)
